# baseline (speedup 1.0000x reference)
import jax
import jax.numpy as jnp
from jax import lax
from jax.experimental import pallas as pl
from jax.experimental.pallas import tpu as pltpu
import jax.experimental.pallas.tpu_sc as plsc

N = 320000
D = 128
NUM_SEG = 16384
NSUB = 16
NW = 32
SEGW = NUM_SEG // NW
R = 128
NSLOT = 3
TRASH = SEGW


def _body(x_hbm, seg_hbm, out_hbm,
          acc, buf0, buf1, buf2, ix0, ix1, ix2, pa, pb, ia, ib,
          sx0, sx1, sx2, si0, si1, si2):
    c = lax.axis_index("c")
    s = lax.axis_index("s")
    w = c * NSUB + s
    seg_lo = w * SEGW
    seg_hi = seg_lo + SEGW
    slots = ((buf0, ix0, sx0, si0),
             (buf1, ix1, sx1, si1),
             (buf2, ix2, sx2, si2))
    lane = lax.iota(jnp.int32, 16)

    zero16 = jnp.zeros((16,), jnp.float32)

    def zrow(r, _):
        for j in range(D // 16):
            acc[r, pl.ds(j * 16, 16)] = zero16
        return 0

    lax.fori_loop(0, SEGW + 1, zrow, 0)

    def count_below(vec, tgt):
        cnt = jnp.int32(0)
        for i in range(16):
            cnt = cnt + jnp.minimum(jnp.maximum(tgt - vec[i], 0), 1)
        return cnt

    def sround(_, st):
        lo0, hi0, lo1, hi1 = st
        q0 = jnp.maximum((hi0 - lo0) // 17, 1)
        q1 = jnp.maximum((hi1 - lo1) // 17, 1)
        ia[...] = jnp.minimum(lo0 + (lane + 1) * q0, N - 1)
        ib[...] = jnp.minimum(lo1 + (lane + 1) * q1, N - 1)
        da = pltpu.async_copy(seg_hbm.at[ia], pa, sx0)
        db = pltpu.async_copy(seg_hbm.at[ib], pb, sx1)
        da.wait()
        db.wait()
        m0 = count_below(pa[...], seg_lo)
        m1 = count_below(pb[...], seg_hi)
        nlo0 = jnp.where(m0 == 0, lo0,
                         jnp.minimum(lo0 + m0 * q0 + 1, N))
        nhi0 = jnp.where(m0 == 16, hi0,
                         jnp.minimum(lo0 + (m0 + 1) * q0, hi0))
        nlo1 = jnp.where(m1 == 0, lo1,
                         jnp.minimum(lo1 + m1 * q1 + 1, N))
        nhi1 = jnp.where(m1 == 16, hi1,
                         jnp.minimum(lo1 + (m1 + 1) * q1, hi1))
        return nlo0, nhi0, nlo1, nhi1

    z = jnp.int32(0)
    n = jnp.int32(N)
    lo0, hi0, lo1, hi1 = lax.fori_loop(0, 5, sround, (z, n, z, n))
    ia[...] = jnp.minimum(lo0 + lane, N - 1)
    ib[...] = jnp.minimum(lo1 + lane, N - 1)
    da = pltpu.async_copy(seg_hbm.at[ia], pa, sx0)
    db = pltpu.async_copy(seg_hbm.at[ib], pb, sx1)
    da.wait()
    db.wait()

    def final_cnt(vec, base, tgt):
        cnt = jnp.int32(0)
        for i in range(16):
            inb = jnp.where(base + i < N, 1, 0)
            cnt = cnt + inb * jnp.minimum(jnp.maximum(tgt - vec[i], 0), 1)
        return cnt

    r0 = lo0 + final_cnt(pa[...], lo0, seg_lo)
    r1 = lo1 + final_cnt(pb[...], lo1, seg_hi)

    a0 = r0 - lax.rem(r0, 8)
    nwin = (r1 - a0 + (R - 1)) // R

    def st_of(k):
        return pl.multiple_of(jnp.minimum(a0 + k * R, N - R), 8)

    def issue_gather(k, slot):
        buf, ix, sx, si = slot

        @pl.when(k < nwin)
        def _():
            st = st_of(k)
            pltpu.async_copy(x_hbm.at[pl.ds(st, R)], buf, sx)
            pltpu.async_copy(seg_hbm.at[pl.ds(st, R)], ix.at[0], si)

    def consume(k, slot):
        buf, ix, sx, si = slot

        @pl.when(k < nwin)
        def _():
            pltpu.make_async_copy(x_hbm.at[pl.ds(0, R)], buf, sx).wait()
            pltpu.make_async_copy(seg_hbm.at[pl.ds(0, R)], ix.at[0],
                                  si).wait()
            st = st_of(k)
            lo_k = jnp.maximum(a0 + k * R, r0)
            hi_k = jnp.minimum(a0 + k * R + R, r1)

            def group(jg, _):
                seg = ix[0, pl.ds(jg * 16, 16)]
                g = st + jg * 16 + lane
                valid = (g >= lo_k) & (g < hi_k)
                li = jnp.where(valid, seg - seg_lo, TRASH)
                for i in range(16):
                    liv = li[i]
                    row = jg * 16 + i
                    for dg in range(D // 16):
                        plsc.addupdate(acc.at[liv, pl.ds(dg * 16, 16)],
                                       buf[row, pl.ds(dg * 16, 16)])
                return 0

            lax.fori_loop(0, R // 16, group, 0)

    issue_gather(jnp.int32(0), slots[0])
    issue_gather(jnp.int32(1), slots[1])

    def outer(g, _):
        for b in range(NSLOT):
            k = g * NSLOT + b
            consume(k, slots[b])
            issue_gather(k + 2, slots[(b - 1) % NSLOT])
        return 0

    lax.fori_loop(0, (nwin + NSLOT - 1) // NSLOT, outer, 0)

    pltpu.sync_copy(acc.at[pl.ds(0, SEGW)],
                    out_hbm.at[pl.ds(pl.multiple_of(w * SEGW, SEGW), SEGW)])


@jax.jit
def _segment_sum(outputs, atom_split):
    mesh = plsc.VectorSubcoreMesh(core_axis_name="c", subcore_axis_name="s")
    return pl.kernel(
        _body,
        out_type=jax.ShapeDtypeStruct((NUM_SEG, D), jnp.float32),
        mesh=mesh,
        scratch_types=[
            pltpu.VMEM((SEGW + 1, D), jnp.float32),
            pltpu.VMEM((R, D), jnp.float32),
            pltpu.VMEM((R, D), jnp.float32),
            pltpu.VMEM((R, D), jnp.float32),
            pltpu.VMEM((1, R), jnp.int32),
            pltpu.VMEM((1, R), jnp.int32),
            pltpu.VMEM((1, R), jnp.int32),
            pltpu.VMEM((16,), jnp.int32),
            pltpu.VMEM((16,), jnp.int32),
            pltpu.VMEM((16,), jnp.int32),
            pltpu.VMEM((16,), jnp.int32),
            pltpu.SemaphoreType.DMA,
            pltpu.SemaphoreType.DMA,
            pltpu.SemaphoreType.DMA,
            pltpu.SemaphoreType.DMA,
            pltpu.SemaphoreType.DMA,
            pltpu.SemaphoreType.DMA,
        ],
    )(outputs, atom_split)


def kernel(outputs, pair_features, atom_split, dummy):
    return (_segment_sum(outputs, atom_split), pair_features)


# speedup vs baseline: 2.2643x; 2.2643x over previous
import jax
import jax.numpy as jnp
from jax import lax
from jax.experimental import pallas as pl
from jax.experimental.pallas import tpu as pltpu
import jax.experimental.pallas.tpu_sc as plsc

N = 320000
D = 128
NUM_SEG = 16384
HALF = NUM_SEG // 2
NSUB = 16
R = 128
NSLOT = 3
NWIN16 = N // 16
TRASH = HALF


def _body(x_hbm, seg_hbm, out_hbm, acc_sh,
          buf0, buf1, buf2, ix0, ix1, ix2, probe, ia,
          sx0, sx1, sx2, si0, si1, si2, ss0, ss1, ss2, sp):
    c = lax.axis_index("c")
    s = lax.axis_index("s")
    slots = ((buf0, ix0, sx0, si0, ss0),
             (buf1, ix1, sx1, si1, ss1),
             (buf2, ix2, sx2, si2, ss2))

    zero16 = jnp.zeros((16,), jnp.float32)

    def zrow(r, _):
        for j in range(D // 16):
            buf0[r, pl.ds(j * 16, 16)] = zero16
        return 0

    lax.fori_loop(0, R, zrow, 0)
    for t in range(512 // R):
        pltpu.sync_copy(
            buf0, acc_sh.at[pl.ds(pl.multiple_of(s * 512 + t * R, R), R)])
    plsc.subcore_barrier()

    lane = lax.iota(jnp.int32, 16)

    def count_below(vec, tgt):
        cnt = jnp.int32(0)
        for i in range(16):
            cnt = cnt + jnp.minimum(jnp.maximum(tgt - vec[i], 0), 1)
        return cnt

    def sround(_, st):
        lo, hi = st
        q = jnp.maximum((hi - lo) // 17, 1)
        ia[...] = jnp.minimum(lo + (lane + 1) * q, N - 1)
        pltpu.async_copy(seg_hbm.at[ia], probe, sp).wait()
        m = count_below(probe[...], HALF)
        nlo = jnp.where(m == 0, lo, jnp.minimum(lo + m * q + 1, N))
        nhi = jnp.where(m == 16, hi, jnp.minimum(lo + (m + 1) * q, hi))
        return nlo, nhi

    lo, hi = lax.fori_loop(0, 5, sround, (jnp.int32(0), jnp.int32(N)))
    ia[...] = jnp.minimum(lo + lane, N - 1)
    pltpu.async_copy(seg_hbm.at[ia], probe, sp).wait()
    pv = probe[...]
    cnt_lo = jnp.int32(0)
    for i in range(16):
        inb = jnp.where(lo + i < N, 1, 0)
        cnt_lo = cnt_lo + inb * jnp.minimum(
            jnp.maximum(HALF - pv[i], 0), 1)
    p_split = (lo + cnt_lo).astype(jnp.int32)

    base = jnp.where(c == 0, 0, p_split)
    limit = jnp.where(c == 0, p_split, N)
    length = limit - base
    r0 = base + (s * length) // NSUB
    r1 = base + ((s + 1) * length) // NSUB
    a0 = r0 - lax.rem(r0, 8)
    nwin = (r1 - a0 + (R - 1)) // R

    seg_base = c * HALF

    def st_of(k):
        return pl.multiple_of(jnp.minimum(a0 + k * R, N - R), 8)

    def issue_gather(k, slot):
        buf, ix, sx, si, _ = slot

        @pl.when(k < nwin)
        def _():
            st = st_of(k)
            pltpu.async_copy(x_hbm.at[pl.ds(st, R)], buf, sx)
            pltpu.async_copy(seg_hbm.at[pl.ds(st, R)], ix.at[0], si)

    def consume(k, slot):
        buf, ix, sx, si, ss = slot

        @pl.when(k < nwin)
        def _():
            pltpu.make_async_copy(x_hbm.at[pl.ds(0, R)], buf, sx).wait()
            pltpu.make_async_copy(seg_hbm.at[pl.ds(0, R)], ix.at[0],
                                  si).wait()
            st = st_of(k)
            lo_k = jnp.maximum(a0 + k * R, r0)
            hi_k = jnp.minimum(a0 + k * R + R, r1)
            for j in range(R // 16):
                seg = ix[0, pl.ds(j * 16, 16)]
                g = st + j * 16 + lane
                valid = (g >= lo_k) & (g < hi_k)
                li = jnp.where(valid, seg - seg_base, TRASH)
                ix[0, pl.ds(j * 16, 16)] = li
            pltpu.async_copy(buf, acc_sh.at[ix.at[0]], ss, add=True)

    def wait_scatter(k, slot):
        buf, ix, _, _, ss = slot

        @pl.when((k >= 0) & (k < nwin))
        def _():
            pltpu.make_async_copy(buf, acc_sh.at[ix.at[0]], ss).wait()

    issue_gather(jnp.int32(0), slots[0])
    issue_gather(jnp.int32(1), slots[1])

    def outer(g, _):
        for b in range(NSLOT):
            k = g * NSLOT + b
            consume(k, slots[b])
            pb = (b - 1) % NSLOT
            wait_scatter(k - 1, slots[pb])
            issue_gather(k + 2, slots[pb])
        return 0

    lax.fori_loop(0, (nwin + NSLOT - 1) // NSLOT, outer, 0)
    @pl.when((lax.rem(nwin, NSLOT) == 0) & (nwin > 0))
    def _():
        buf, ix, _, _, ss = slots[NSLOT - 1]
        pltpu.make_async_copy(buf, acc_sh.at[ix.at[0]], ss).wait()

    plsc.subcore_barrier()

    pltpu.sync_copy(acc_sh.at[pl.ds(pl.multiple_of(s * 512, 512), 512)],
                    out_hbm.at[pl.ds(pl.multiple_of(c * HALF + s * 512, 512),
                                     512)])


@jax.jit
def _segment_sum(outputs, atom_split):
    mesh = plsc.VectorSubcoreMesh(core_axis_name="c", subcore_axis_name="s")
    return pl.kernel(
        _body,
        out_type=jax.ShapeDtypeStruct((NUM_SEG, D), jnp.float32),
        mesh=mesh,
        scratch_types=[
            pltpu.MemorySpace.VMEM_SHARED((HALF + 1, D), jnp.float32),
            pltpu.VMEM((R, D), jnp.float32),
            pltpu.VMEM((R, D), jnp.float32),
            pltpu.VMEM((R, D), jnp.float32),
            pltpu.VMEM((1, R), jnp.int32),
            pltpu.VMEM((1, R), jnp.int32),
            pltpu.VMEM((1, R), jnp.int32),
            pltpu.VMEM((16,), jnp.int32),
            pltpu.VMEM((16,), jnp.int32),
            pltpu.SemaphoreType.DMA,
            pltpu.SemaphoreType.DMA,
            pltpu.SemaphoreType.DMA,
            pltpu.SemaphoreType.DMA,
            pltpu.SemaphoreType.DMA,
            pltpu.SemaphoreType.DMA,
            pltpu.SemaphoreType.DMA,
            pltpu.SemaphoreType.DMA,
            pltpu.SemaphoreType.DMA,
            pltpu.SemaphoreType.DMA,
        ],
    )(outputs, atom_split)


def kernel(outputs, pair_features, atom_split, dummy):
    return (_segment_sum(outputs, atom_split), pair_features)


# speedup vs baseline: 2.2753x; 1.0048x over previous
import jax
import jax.numpy as jnp
from jax import lax
from jax.experimental import pallas as pl
from jax.experimental.pallas import tpu as pltpu
import jax.experimental.pallas.tpu_sc as plsc

N = 320000
D = 128
NUM_SEG = 16384
HALF = NUM_SEG // 2
NSUB = 16
R = 128
NSLOT = 3
NWIN16 = N // 16
TRASH = HALF


def _body(x_hbm, seg_hbm, out_hbm, acc_sh,
          buf0, buf1, buf2, ixa0, ixb0, ixa1, ixb1, ixa2, ixb2, probe,
          sx0, sx1, sx2, si0, si1, si2, ss0, ss1, ss2):
    c = lax.axis_index("c")
    s = lax.axis_index("s")
    slots = ((buf0, ixa0, ixb0, sx0, si0, ss0),
             (buf1, ixa1, ixb1, sx1, si1, ss1),
             (buf2, ixa2, ixb2, sx2, si2, ss2))

    zero16 = jnp.zeros((16,), jnp.float32)

    def zrow(r, _):
        for j in range(D // 16):
            buf0[r, pl.ds(j * 16, 16)] = zero16
        return 0

    lax.fori_loop(0, R, zrow, 0)
    for t in range(512 // R):
        pltpu.sync_copy(
            buf0, acc_sh.at[pl.ds(pl.multiple_of(s * 512 + t * R, R), R)])
    plsc.subcore_barrier()

    def probe_win(w):
        pltpu.sync_copy(seg_hbm.at[pl.ds(pl.multiple_of(w * 16, 16), 16)],
                        probe)

    def bstep(_, st):
        lo, hi = st
        active = (hi - lo) > 1
        mid = lo + (hi - lo) // 2
        probe_win(jnp.maximum(mid, 0))
        pred = probe[...][0] < HALF
        take = active & pred
        lo2 = jnp.where(take, mid, lo)
        hi2 = jnp.where(active & (~pred), mid, hi)
        return lo2, hi2

    lo, hi = lax.fori_loop(
        0, 15, bstep, (jnp.int32(-1), jnp.int32(NWIN16)))
    probe_win(jnp.maximum(lo, 0))
    pv = probe[...]
    cnt_lo = jnp.int32(0)
    for i in range(16):
        cnt_lo = cnt_lo + jnp.minimum(
            jnp.maximum(HALF - pv[i], 0), 1)
    p_split = jnp.where(lo < 0, 0, lo * 16 + cnt_lo).astype(jnp.int32)

    base = jnp.where(c == 0, 0, p_split)
    limit = jnp.where(c == 0, p_split, N)
    length = limit - base
    r0 = base + (s * length) // NSUB
    r1 = base + ((s + 1) * length) // NSUB
    a0 = r0 - lax.rem(r0, 8)
    nwin = (r1 - a0 + (R - 1)) // R

    seg_base = c * HALF
    lane = lax.iota(jnp.int32, 16)

    def st_of(k):
        return pl.multiple_of(jnp.minimum(a0 + k * R, N - R), 8)

    def issue_gather(k, slot):
        buf, ixa, ixb, sx, si, _ = slot

        @pl.when(k < nwin)
        def _():
            st = st_of(k)
            pltpu.async_copy(x_hbm.at[pl.ds(st, R)], buf, sx)
            pltpu.async_copy(seg_hbm.at[pl.ds(st, R // 2)], ixa, si)
            pltpu.async_copy(
                seg_hbm.at[pl.ds(pl.multiple_of(st + R // 2, 8), R // 2)],
                ixb, si)

    def consume(k, slot):
        buf, ixa, ixb, sx, si, ss = slot

        @pl.when(k < nwin)
        def _():
            pltpu.make_async_copy(x_hbm.at[pl.ds(0, R)], buf, sx).wait()
            pltpu.make_async_copy(seg_hbm.at[pl.ds(0, R // 2)], ixa,
                                  si).wait()
            pltpu.make_async_copy(seg_hbm.at[pl.ds(0, R // 2)], ixb,
                                  si).wait()
            st = st_of(k)
            lo_k = jnp.maximum(a0 + k * R, r0)
            hi_k = jnp.minimum(a0 + k * R + R, r1)
            for j in range(R // 16):
                ix = ixa if j < (R // 32) else ixb
                off = (j * 16) % (R // 2)
                seg = ix[pl.ds(off, 16)]
                g = st + j * 16 + lane
                valid = (g >= lo_k) & (g < hi_k)
                li = jnp.where(valid, seg - seg_base, TRASH)
                ix[pl.ds(off, 16)] = li
            pltpu.async_copy(buf.at[pl.ds(0, R // 2)], acc_sh.at[ixa],
                             ss, add=True)
            pltpu.async_copy(buf.at[pl.ds(R // 2, R // 2)], acc_sh.at[ixb],
                             ss, add=True)

    def wait_scatter(k, slot):
        buf, ixa, ixb, _, _, ss = slot

        @pl.when((k >= 0) & (k < nwin))
        def _():
            pltpu.make_async_copy(buf.at[pl.ds(0, R // 2)], acc_sh.at[ixa],
                                  ss).wait()
            pltpu.make_async_copy(buf.at[pl.ds(R // 2, R // 2)],
                                  acc_sh.at[ixb], ss).wait()

    issue_gather(jnp.int32(0), slots[0])
    issue_gather(jnp.int32(1), slots[1])

    def outer(g, _):
        for b in range(NSLOT):
            k = g * NSLOT + b
            consume(k, slots[b])
            pb = (b - 1) % NSLOT
            wait_scatter(k - 1, slots[pb])
            issue_gather(k + 2, slots[pb])
        return 0

    lax.fori_loop(0, (nwin + NSLOT - 1) // NSLOT, outer, 0)
    @pl.when((lax.rem(nwin, NSLOT) == 0) & (nwin > 0))
    def _():
        buf, ixa, ixb, _, _, ss = slots[NSLOT - 1]
        pltpu.make_async_copy(buf.at[pl.ds(0, R // 2)], acc_sh.at[ixa],
                              ss).wait()
        pltpu.make_async_copy(buf.at[pl.ds(R // 2, R // 2)],
                              acc_sh.at[ixb], ss).wait()

    plsc.subcore_barrier()

    pltpu.sync_copy(acc_sh.at[pl.ds(pl.multiple_of(s * 512, 512), 512)],
                    out_hbm.at[pl.ds(pl.multiple_of(c * HALF + s * 512, 512),
                                     512)])


@jax.jit
def _segment_sum(outputs, atom_split):
    mesh = plsc.VectorSubcoreMesh(core_axis_name="c", subcore_axis_name="s")
    return pl.kernel(
        _body,
        out_type=jax.ShapeDtypeStruct((NUM_SEG, D), jnp.float32),
        mesh=mesh,
        scratch_types=[
            pltpu.MemorySpace.VMEM_SHARED((HALF + 1, D), jnp.float32),
            pltpu.VMEM((R, D), jnp.float32),
            pltpu.VMEM((R, D), jnp.float32),
            pltpu.VMEM((R, D), jnp.float32),
            pltpu.VMEM((R // 2,), jnp.int32),
            pltpu.VMEM((R // 2,), jnp.int32),
            pltpu.VMEM((R // 2,), jnp.int32),
            pltpu.VMEM((R // 2,), jnp.int32),
            pltpu.VMEM((R // 2,), jnp.int32),
            pltpu.VMEM((R // 2,), jnp.int32),
            pltpu.VMEM((16,), jnp.int32),
            pltpu.SemaphoreType.DMA,
            pltpu.SemaphoreType.DMA,
            pltpu.SemaphoreType.DMA,
            pltpu.SemaphoreType.DMA,
            pltpu.SemaphoreType.DMA,
            pltpu.SemaphoreType.DMA,
            pltpu.SemaphoreType.DMA,
            pltpu.SemaphoreType.DMA,
            pltpu.SemaphoreType.DMA,
        ],
    )(outputs, atom_split)


def kernel(outputs, pair_features, atom_split, dummy):
    return (_segment_sum(outputs, atom_split), pair_features)


# speedup vs baseline: 2.3230x; 1.0210x over previous
import jax
import jax.numpy as jnp
from jax import lax
from jax.experimental import pallas as pl
from jax.experimental.pallas import tpu as pltpu
import jax.experimental.pallas.tpu_sc as plsc

N = 320000
D = 128
NUM_SEG = 16384
HALF = NUM_SEG // 2
NSUB = 16
R = 128
NSLOT = 3
NWIN16 = N // 16
TRASH = HALF


def _body(x_hbm, seg_hbm, out_hbm, acc_sh,
          buf0, buf1, buf2, ix0, ix1, ix2, probe,
          sx0, sx1, sx2, si0, si1, si2, ss0, ss1, ss2):
    c = lax.axis_index("c")
    s = lax.axis_index("s")
    slots = ((buf0, ix0, sx0, si0, ss0),
             (buf1, ix1, sx1, si1, ss1),
             (buf2, ix2, sx2, si2, ss2))

    zero16 = jnp.zeros((16,), jnp.float32)

    def zrow(r, _):
        for j in range(D // 16):
            buf0[r, pl.ds(j * 16, 16)] = zero16
        return 0

    lax.fori_loop(0, R, zrow, 0)
    zd = []
    for t in range(512 // R):
        zd.append(pltpu.async_copy(
            buf0, acc_sh.at[pl.ds(pl.multiple_of(s * 512 + t * R, R), R)],
            ss0))

    def probe_win(w):
        pltpu.sync_copy(seg_hbm.at[pl.ds(pl.multiple_of(w * 16, 16), 16)],
                        probe)

    def bstep(_, st):
        lo, hi = st
        active = (hi - lo) > 1
        mid = lo + (hi - lo) // 2
        probe_win(jnp.maximum(mid, 0))
        pred = probe[...][0] < HALF
        take = active & pred
        lo2 = jnp.where(take, mid, lo)
        hi2 = jnp.where(active & (~pred), mid, hi)
        return lo2, hi2

    lo, hi = lax.fori_loop(
        0, 15, bstep, (jnp.int32(-1), jnp.int32(NWIN16)))
    probe_win(jnp.maximum(lo, 0))
    pv = probe[...]
    cnt_lo = jnp.int32(0)
    for i in range(16):
        cnt_lo = cnt_lo + jnp.minimum(
            jnp.maximum(HALF - pv[i], 0), 1)
    p_split = jnp.where(lo < 0, 0, lo * 16 + cnt_lo).astype(jnp.int32)

    base = jnp.where(c == 0, 0, p_split)
    limit = jnp.where(c == 0, p_split, N)
    length = limit - base
    r0 = base + (s * length) // NSUB
    r1 = base + ((s + 1) * length) // NSUB
    a0 = r0 - lax.rem(r0, 8)
    nwin = (r1 - a0 + (R - 1)) // R

    seg_base = c * HALF
    lane = lax.iota(jnp.int32, 16)

    def st_of(k):
        return pl.multiple_of(jnp.minimum(a0 + k * R, N - R), 8)

    def issue_gather(k, slot):
        buf, ix, sx, si, _ = slot

        @pl.when(k < nwin)
        def _():
            st = st_of(k)
            pltpu.async_copy(x_hbm.at[pl.ds(st, R)], buf, sx)
            pltpu.async_copy(seg_hbm.at[pl.ds(st, R)], ix.at[0], si)

    def consume(k, slot):
        buf, ix, sx, si, ss = slot

        @pl.when(k < nwin)
        def _():
            pltpu.make_async_copy(x_hbm.at[pl.ds(0, R)], buf, sx).wait()
            pltpu.make_async_copy(seg_hbm.at[pl.ds(0, R)], ix.at[0],
                                  si).wait()
            st = st_of(k)
            lo_k = jnp.maximum(a0 + k * R, r0)
            hi_k = jnp.minimum(a0 + k * R + R, r1)
            for j in range(R // 16):
                seg = ix[0, pl.ds(j * 16, 16)]
                g = st + j * 16 + lane
                valid = (g >= lo_k) & (g < hi_k)
                li = jnp.where(valid, seg - seg_base, TRASH)
                ix[0, pl.ds(j * 16, 16)] = li
            pltpu.async_copy(buf, acc_sh.at[ix.at[0]], ss, add=True)

    def wait_scatter(k, slot):
        buf, ix, _, _, ss = slot

        @pl.when((k >= 0) & (k < nwin))
        def _():
            pltpu.make_async_copy(buf, acc_sh.at[ix.at[0]], ss).wait()

    for d in zd:
        d.wait()
    issue_gather(jnp.int32(0), slots[0])
    issue_gather(jnp.int32(1), slots[1])
    plsc.subcore_barrier()

    def outer(g, _):
        for b in range(NSLOT):
            k = g * NSLOT + b
            consume(k, slots[b])
            pb = (b - 1) % NSLOT
            wait_scatter(k - 1, slots[pb])
            issue_gather(k + 2, slots[pb])
        return 0

    lax.fori_loop(0, (nwin + NSLOT - 1) // NSLOT, outer, 0)
    @pl.when((lax.rem(nwin, NSLOT) == 0) & (nwin > 0))
    def _():
        buf, ix, _, _, ss = slots[NSLOT - 1]
        pltpu.make_async_copy(buf, acc_sh.at[ix.at[0]], ss).wait()

    plsc.subcore_barrier()

    pltpu.sync_copy(acc_sh.at[pl.ds(pl.multiple_of(s * 512, 512), 512)],
                    out_hbm.at[pl.ds(pl.multiple_of(c * HALF + s * 512, 512),
                                     512)])


@jax.jit
def _segment_sum(outputs, atom_split):
    mesh = plsc.VectorSubcoreMesh(core_axis_name="c", subcore_axis_name="s")
    return pl.kernel(
        _body,
        out_type=jax.ShapeDtypeStruct((NUM_SEG, D), jnp.float32),
        mesh=mesh,
        scratch_types=[
            pltpu.MemorySpace.VMEM_SHARED((HALF + 1, D), jnp.float32),
            pltpu.VMEM((R, D), jnp.float32),
            pltpu.VMEM((R, D), jnp.float32),
            pltpu.VMEM((R, D), jnp.float32),
            pltpu.VMEM((1, R), jnp.int32),
            pltpu.VMEM((1, R), jnp.int32),
            pltpu.VMEM((1, R), jnp.int32),
            pltpu.VMEM((16,), jnp.int32),
            pltpu.SemaphoreType.DMA,
            pltpu.SemaphoreType.DMA,
            pltpu.SemaphoreType.DMA,
            pltpu.SemaphoreType.DMA,
            pltpu.SemaphoreType.DMA,
            pltpu.SemaphoreType.DMA,
            pltpu.SemaphoreType.DMA,
            pltpu.SemaphoreType.DMA,
            pltpu.SemaphoreType.DMA,
        ],
    )(outputs, atom_split)


def kernel(outputs, pair_features, atom_split, dummy):
    return (_segment_sum(outputs, atom_split), pair_features)
